# baseline (speedup 1.0000x reference)
import jax
import jax.numpy as jnp
from jax import lax
from jax.experimental import pallas as pl
from jax.experimental.pallas import tpu as pltpu


def _pass1_kernel(L_ref, x0_ref, x1_ref, px2_ref, x1_acc):
    bm = L_ref.shape[0]
    i = pl.program_id(0)
    x1_i = jnp.dot(L_ref[...], x0_ref[...], preferred_element_type=jnp.float32)
    x1_ref[...] = x1_i
    x1_acc[pl.ds(i * bm, bm), :] = x1_i

    def body(j, acc):
        lb = L_ref[:, pl.ds(j * bm, bm)]
        xb = x1_acc[pl.ds(j * bm, bm), :]
        return acc + jnp.dot(lb, xb, preferred_element_type=jnp.float32)

    acc0 = jnp.zeros((bm, x0_ref.shape[1]), jnp.float32)
    acc = lax.fori_loop(0, i + 1, body, acc0)
    px2_ref[...] = 2.0 * acc


def _pass2_kernel(L_ref, x1_ref, px2_ref, x0b_ref, WA_ref, WB_ref, WC_ref,
                  bias_ref, o_ref, acc):
    bm = L_ref.shape[0]
    nb = pl.num_programs(1)
    i = pl.program_id(0)
    jb = pl.program_id(1)

    @pl.when(jb == 0)
    def _init():
        acc[...] = px2_ref[...]

    @pl.when(jb > i)
    def _accum():
        xb = x1_ref[pl.ds(jb * bm, bm), :]
        acc[...] += 2.0 * jnp.dot(L_ref[...], xb,
                                  preferred_element_type=jnp.float32)

    @pl.when(jb == nb - 1)
    def _finish():
        x1b = x1_ref[pl.ds(i * bm, bm), :]
        o_ref[...] = (
            jnp.dot(x0b_ref[...], WA_ref[...],
                    preferred_element_type=jnp.float32)
            + jnp.dot(x1b, WB_ref[...], preferred_element_type=jnp.float32)
            + jnp.dot(acc[...], WC_ref[...],
                      preferred_element_type=jnp.float32)
            + bias_ref[...]
        )


def kernel(x, L, weight, bias):
    N, M, Fin = x.shape
    Fout = weight.shape[1]
    x0 = jnp.transpose(x, (1, 0, 2)).reshape(M, N * Fin)

    W = weight.reshape(Fin, 3, Fout)
    eyeN = jnp.eye(N, dtype=weight.dtype)
    W0e = jnp.kron(eyeN, W[:, 0, :])
    W1e = jnp.kron(eyeN, W[:, 1, :])
    W2e = jnp.kron(eyeN, W[:, 2, :])
    WA = W0e - W2e
    WB = W1e
    WC = W2e
    bias_row = jnp.tile(bias, N).reshape(1, N * Fout)

    bm = 256
    nb = M // bm
    C = N * Fin
    Co = N * Fout

    x1, px2 = pl.pallas_call(
        _pass1_kernel,
        grid=(nb,),
        in_specs=[
            pl.BlockSpec((bm, M), lambda i: (i, 0)),
            pl.BlockSpec((M, C), lambda i: (0, 0)),
        ],
        out_specs=[
            pl.BlockSpec((bm, C), lambda i: (i, 0)),
            pl.BlockSpec((bm, C), lambda i: (i, 0)),
        ],
        out_shape=[
            jax.ShapeDtypeStruct((M, C), jnp.float32),
            jax.ShapeDtypeStruct((M, C), jnp.float32),
        ],
        scratch_shapes=[pltpu.VMEM((M, C), jnp.float32)],
    )(L, x0)

    def _L_index(i, jb):
        return (i, jnp.minimum(jnp.maximum(jb, i + 1), nb - 1))

    out_flat = pl.pallas_call(
        _pass2_kernel,
        grid=(nb, nb),
        in_specs=[
            pl.BlockSpec((bm, bm), _L_index),
            pl.BlockSpec((M, C), lambda i, jb: (0, 0)),
            pl.BlockSpec((bm, C), lambda i, jb: (i, 0)),
            pl.BlockSpec((bm, C), lambda i, jb: (i, 0)),
            pl.BlockSpec((C, Co), lambda i, jb: (0, 0)),
            pl.BlockSpec((C, Co), lambda i, jb: (0, 0)),
            pl.BlockSpec((C, Co), lambda i, jb: (0, 0)),
            pl.BlockSpec((1, Co), lambda i, jb: (0, 0)),
        ],
        out_specs=pl.BlockSpec((bm, Co), lambda i, jb: (i, 0)),
        out_shape=jax.ShapeDtypeStruct((M, Co), jnp.float32),
        scratch_shapes=[pltpu.VMEM((bm, C), jnp.float32)],
    )(L, x1, px2, x0, WA, WB, WC, bias_row)

    return out_flat.reshape(M, N, Fout).transpose(1, 0, 2)


# speedup vs baseline: 2.8170x; 2.8170x over previous
import jax
import jax.numpy as jnp
from jax.experimental import pallas as pl


def _pass1_kernel(L_ref, x0_ref, x1_ref):
    x1_ref[...] = jnp.dot(L_ref[...].astype(jnp.bfloat16), x0_ref[...],
                          preferred_element_type=jnp.float32)


def _pass2_kernel(L_ref, x1_ref, x0b_ref, x1b_ref, WA_ref, WB_ref, WC_ref,
                  bias_ref, o_ref):
    t = jnp.dot(L_ref[...].astype(jnp.bfloat16),
                x1_ref[...].astype(jnp.bfloat16),
                preferred_element_type=jnp.float32)
    o_ref[...] = (
        jnp.dot(x0b_ref[...], WA_ref[...], preferred_element_type=jnp.float32)
        + jnp.dot(x1b_ref[...], WB_ref[...], preferred_element_type=jnp.float32)
        + jnp.dot(t, WC_ref[...], preferred_element_type=jnp.float32)
        + bias_ref[...]
    )


def kernel(x, L, weight, bias):
    N, M, Fin = x.shape
    Fout = weight.shape[1]
    x0 = jnp.transpose(x, (1, 0, 2)).reshape(M, N * Fin)
    x0bf = x0.astype(jnp.bfloat16)

    W = weight.reshape(Fin, 3, Fout)
    eyeN = jnp.eye(N, dtype=weight.dtype)
    W0e = jnp.kron(eyeN, W[:, 0, :])
    W1e = jnp.kron(eyeN, W[:, 1, :])
    W2e = jnp.kron(eyeN, W[:, 2, :])
    WA = W0e - W2e
    WB = W1e
    WC = 2.0 * W2e
    bias_row = jnp.tile(bias, N).reshape(1, N * Fout)

    bm = 256
    C = N * Fin
    Co = N * Fout

    x1 = pl.pallas_call(
        _pass1_kernel,
        grid=(M // bm,),
        in_specs=[
            pl.BlockSpec((bm, M), lambda i: (i, 0)),
            pl.BlockSpec((M, C), lambda i: (0, 0)),
        ],
        out_specs=pl.BlockSpec((bm, C), lambda i: (i, 0)),
        out_shape=jax.ShapeDtypeStruct((M, C), jnp.float32),
    )(L, x0bf)

    out_flat = pl.pallas_call(
        _pass2_kernel,
        grid=(M // bm,),
        in_specs=[
            pl.BlockSpec((bm, M), lambda i: (i, 0)),
            pl.BlockSpec((M, C), lambda i: (0, 0)),
            pl.BlockSpec((bm, C), lambda i: (i, 0)),
            pl.BlockSpec((bm, C), lambda i: (i, 0)),
            pl.BlockSpec((C, Co), lambda i: (0, 0)),
            pl.BlockSpec((C, Co), lambda i: (0, 0)),
            pl.BlockSpec((C, Co), lambda i: (0, 0)),
            pl.BlockSpec((1, Co), lambda i: (0, 0)),
        ],
        out_specs=pl.BlockSpec((bm, Co), lambda i: (i, 0)),
        out_shape=jax.ShapeDtypeStruct((M, Co), jnp.float32),
    )(L, x1, x0, x1, WA, WB, WC, bias_row)

    return out_flat.reshape(M, N, Fout).transpose(1, 0, 2)


# speedup vs baseline: 2.8962x; 1.0281x over previous
import jax
import jax.numpy as jnp
from jax.experimental import pallas as pl
from jax.experimental.pallas import tpu as pltpu


def _pass1_kernel(L_ref, x0bf_ref, x1bf_ref, px2_ref, xcat, xacc, Ldiag):
    nb = pl.num_programs(1)
    i = pl.program_id(0)
    jb = pl.program_id(1)
    bm = L_ref.shape[0]
    C = x0bf_ref.shape[1]

    @pl.when(i == 0)
    def _seed_x0():
        xcat[pl.ds(jb * bm, bm), :C] = x0bf_ref[pl.ds(jb * bm, bm), :]

    Lb = L_ref[...].astype(jnp.bfloat16)
    rhs = xcat[pl.ds(jb * bm, bm), :]
    prod = jnp.dot(Lb, rhs, preferred_element_type=jnp.float32)

    @pl.when(jb == 0)
    def _init():
        xacc[...] = prod[:, :C]
        px2_ref[...] = jnp.zeros_like(px2_ref)

    @pl.when(jb > 0)
    def _x1_accum():
        xacc[...] += prod[:, :C]

    @pl.when(jb < i)
    def _lower():
        px2_ref[...] += prod[:, C:]

    @pl.when(jb == i)
    def _stash_diag():
        Ldiag[...] = Lb

    @pl.when(jb == nb - 1)
    def _finish_strip():
        x1b16 = xacc[...].astype(jnp.bfloat16)
        x1bf_ref[...] = x1b16
        xcat[pl.ds(i * bm, bm), C:] = x1b16
        px2_ref[...] = 2.0 * (
            px2_ref[...]
            + jnp.dot(Ldiag[...], x1b16, preferred_element_type=jnp.float32)
        )


def _pass2_kernel(L_ref, x1bf_ref, px2_ref, x0b_ref, WA_ref, WB_ref, WC_ref,
                  bias_ref, o_ref, acc):
    nb = pl.num_programs(1)
    i = pl.program_id(0)
    jb = pl.program_id(1)
    bm = L_ref.shape[0]

    @pl.when(jb == 0)
    def _init():
        acc[...] = px2_ref[...]

    @pl.when(jb > i)
    def _accum():
        xb = x1bf_ref[pl.ds(jb * bm, bm), :]
        acc[...] += 2.0 * jnp.dot(L_ref[...].astype(jnp.bfloat16), xb,
                                  preferred_element_type=jnp.float32)

    @pl.when(jb == nb - 1)
    def _finish():
        x1b = x1bf_ref[pl.ds(i * bm, bm), :].astype(jnp.float32)
        o_ref[...] = (
            jnp.dot(x0b_ref[...], WA_ref[...],
                    preferred_element_type=jnp.float32)
            + jnp.dot(x1b, WB_ref[...], preferred_element_type=jnp.float32)
            + jnp.dot(acc[...], WC_ref[...],
                      preferred_element_type=jnp.float32)
            + bias_ref[...]
        )


def kernel(x, L, weight, bias):
    N, M, Fin = x.shape
    Fout = weight.shape[1]
    x0 = jnp.transpose(x, (1, 0, 2)).reshape(M, N * Fin)
    x0bf = x0.astype(jnp.bfloat16)

    W = weight.reshape(Fin, 3, Fout)
    eyeN = jnp.eye(N, dtype=weight.dtype)
    W0e = jnp.kron(eyeN, W[:, 0, :])
    W1e = jnp.kron(eyeN, W[:, 1, :])
    W2e = jnp.kron(eyeN, W[:, 2, :])
    WA = W0e - W2e
    WB = W1e
    WC = W2e
    bias_row = jnp.tile(bias, N).reshape(1, N * Fout)

    bm = 1024
    nb = M // bm
    C = N * Fin
    Co = N * Fout

    x1bf, px2 = pl.pallas_call(
        _pass1_kernel,
        grid=(nb, nb),
        in_specs=[
            pl.BlockSpec((bm, bm), lambda i, jb: (i, jb)),
            pl.BlockSpec((M, C), lambda i, jb: (0, 0)),
        ],
        out_specs=[
            pl.BlockSpec((bm, C), lambda i, jb: (i, 0)),
            pl.BlockSpec((bm, C), lambda i, jb: (i, 0)),
        ],
        out_shape=[
            jax.ShapeDtypeStruct((M, C), jnp.bfloat16),
            jax.ShapeDtypeStruct((M, C), jnp.float32),
        ],
        scratch_shapes=[
            pltpu.VMEM((M, 2 * C), jnp.bfloat16),
            pltpu.VMEM((bm, C), jnp.float32),
            pltpu.VMEM((bm, bm), jnp.bfloat16),
        ],
    )(L, x0bf)

    def _L_index(i, jb):
        row = jnp.minimum(i, nb - 2)
        col = jnp.where(i == nb - 1, nb - 1,
                        jnp.minimum(jnp.maximum(jb, i + 1), nb - 1))
        return (row, col)

    out_flat = pl.pallas_call(
        _pass2_kernel,
        grid=(nb, nb),
        in_specs=[
            pl.BlockSpec((bm, bm), _L_index),
            pl.BlockSpec((M, C), lambda i, jb: (0, 0)),
            pl.BlockSpec((bm, C), lambda i, jb: (i, 0)),
            pl.BlockSpec((bm, C), lambda i, jb: (i, 0)),
            pl.BlockSpec((C, Co), lambda i, jb: (0, 0)),
            pl.BlockSpec((C, Co), lambda i, jb: (0, 0)),
            pl.BlockSpec((C, Co), lambda i, jb: (0, 0)),
            pl.BlockSpec((1, Co), lambda i, jb: (0, 0)),
        ],
        out_specs=pl.BlockSpec((bm, Co), lambda i, jb: (i, 0)),
        out_shape=jax.ShapeDtypeStruct((M, Co), jnp.float32),
        scratch_shapes=[pltpu.VMEM((bm, C), jnp.float32)],
    )(L, x1bf, px2, x0, WA, WB, WC, bias_row)

    return out_flat.reshape(M, N, Fout).transpose(1, 0, 2)


# speedup vs baseline: 3.5229x; 1.2164x over previous
import jax
import jax.numpy as jnp
from jax.experimental import pallas as pl
from jax.experimental.pallas import tpu as pltpu


def _p1_col(i, jb, nb):
    shifted = jb + (jb >= i).astype(jb.dtype)
    return jnp.where(jb == nb - 1, i, shifted)


def _pass1_kernel(L_ref, x0bf_ref, x1bf_ref, px2_ref, xcat, xacc):
    nb = pl.num_programs(1)
    i = pl.program_id(0)
    jb = pl.program_id(1)
    bm = L_ref.shape[0]
    C = x0bf_ref.shape[1]
    c = _p1_col(i, jb, nb)

    @pl.when(i == 0)
    def _seed_x0():
        xcat[pl.ds(c * bm, bm), :C] = x0bf_ref[pl.ds(c * bm, bm), :]

    Lb = L_ref[...].astype(jnp.bfloat16)
    rhs = xcat[pl.ds(c * bm, bm), :]
    prod = jnp.dot(Lb, rhs, preferred_element_type=jnp.float32)

    @pl.when(jb == 0)
    def _init():
        xacc[...] = prod[:, :C]
        px2_ref[...] = jnp.zeros_like(px2_ref)

    @pl.when(jb > 0)
    def _x1_accum():
        xacc[...] += prod[:, :C]

    @pl.when(c < i)
    def _lower():
        px2_ref[...] += prod[:, C:]

    @pl.when(jb == nb - 1)
    def _finish_strip():
        x1b16 = xacc[...].astype(jnp.bfloat16)
        x1bf_ref[...] = x1b16
        xcat[pl.ds(i * bm, bm), C:] = x1b16
        px2_ref[...] = 2.0 * (
            px2_ref[...]
            + jnp.dot(Lb, x1b16, preferred_element_type=jnp.float32)
        )


def _pass2_kernel(L_ref, x1bf_ref, px2_ref, x0b_ref, WA_ref, WB_ref, WC_ref,
                  bias_ref, o_ref, acc):
    nb = pl.num_programs(1)
    i = pl.program_id(0)
    jb = pl.program_id(1)
    bm = L_ref.shape[0]

    @pl.when(jb == 0)
    def _init():
        acc[...] = px2_ref[...]

    @pl.when(jb > i)
    def _accum():
        xb = x1bf_ref[pl.ds(jb * bm, bm), :]
        acc[...] += 2.0 * jnp.dot(L_ref[...].astype(jnp.bfloat16), xb,
                                  preferred_element_type=jnp.float32)

    @pl.when(jb == nb - 1)
    def _finish():
        x1b = x1bf_ref[pl.ds(i * bm, bm), :].astype(jnp.float32)
        o_ref[...] = (
            jnp.dot(x0b_ref[...], WA_ref[...],
                    preferred_element_type=jnp.float32)
            + jnp.dot(x1b, WB_ref[...], preferred_element_type=jnp.float32)
            + jnp.dot(acc[...], WC_ref[...],
                      preferred_element_type=jnp.float32)
            + bias_ref[...]
        )


def kernel(x, L, weight, bias):
    N, M, Fin = x.shape
    Fout = weight.shape[1]
    x0 = jnp.transpose(x, (1, 0, 2)).reshape(M, N * Fin)
    x0bf = x0.astype(jnp.bfloat16)

    W = weight.reshape(Fin, 3, Fout)
    eyeN = jnp.eye(N, dtype=weight.dtype)
    W0e = jnp.kron(eyeN, W[:, 0, :])
    W1e = jnp.kron(eyeN, W[:, 1, :])
    W2e = jnp.kron(eyeN, W[:, 2, :])
    WA = W0e - W2e
    WB = W1e
    WC = W2e
    bias_row = jnp.tile(bias, N).reshape(1, N * Fout)

    bm = 2048
    nb = M // bm
    C = N * Fin
    Co = N * Fout

    x1bf, px2 = pl.pallas_call(
        _pass1_kernel,
        grid=(nb, nb),
        in_specs=[
            pl.BlockSpec((bm, bm), lambda i, jb: (i, _p1_col(i, jb, nb))),
            pl.BlockSpec((M, C), lambda i, jb: (0, 0)),
        ],
        out_specs=[
            pl.BlockSpec((bm, C), lambda i, jb: (i, 0)),
            pl.BlockSpec((bm, C), lambda i, jb: (i, 0)),
        ],
        out_shape=[
            jax.ShapeDtypeStruct((M, C), jnp.bfloat16),
            jax.ShapeDtypeStruct((M, C), jnp.float32),
        ],
        scratch_shapes=[
            pltpu.VMEM((M, 2 * C), jnp.bfloat16),
            pltpu.VMEM((bm, C), jnp.float32),
        ],
    )(L, x0bf)

    def _L_index(i, jb):
        row = jnp.minimum(i, nb - 2)
        col = jnp.where(i == nb - 1, nb - 1,
                        jnp.minimum(jnp.maximum(jb, i + 1), nb - 1))
        return (row, col)

    out_flat = pl.pallas_call(
        _pass2_kernel,
        grid=(nb, nb),
        in_specs=[
            pl.BlockSpec((bm, bm), _L_index),
            pl.BlockSpec((M, C), lambda i, jb: (0, 0)),
            pl.BlockSpec((bm, C), lambda i, jb: (i, 0)),
            pl.BlockSpec((bm, C), lambda i, jb: (i, 0)),
            pl.BlockSpec((C, Co), lambda i, jb: (0, 0)),
            pl.BlockSpec((C, Co), lambda i, jb: (0, 0)),
            pl.BlockSpec((C, Co), lambda i, jb: (0, 0)),
            pl.BlockSpec((1, Co), lambda i, jb: (0, 0)),
        ],
        out_specs=pl.BlockSpec((bm, Co), lambda i, jb: (i, 0)),
        out_shape=jax.ShapeDtypeStruct((M, Co), jnp.float32),
        scratch_shapes=[pltpu.VMEM((bm, C), jnp.float32)],
    )(L, x1bf, px2, x0, WA, WB, WC, bias_row)

    return out_flat.reshape(M, N, Fout).transpose(1, 0, 2)


# speedup vs baseline: 3.5348x; 1.0034x over previous
import jax
import jax.numpy as jnp
from jax.experimental import pallas as pl
from jax.experimental.pallas import tpu as pltpu


def _p1_col(i, jb, nb):
    shifted = jb + (jb >= i).astype(jb.dtype)
    return jnp.where(jb == nb - 1, i, shifted)


def _pass1_kernel(L_ref, x0bf_ref, x1f_ref, px2_ref, xcat, xacc):
    nb = pl.num_programs(1)
    i = pl.program_id(0)
    jb = pl.program_id(1)
    bm = L_ref.shape[0]
    C = x0bf_ref.shape[1]
    c = _p1_col(i, jb, nb)

    @pl.when(i == 0)
    def _seed_x0():
        xcat[pl.ds(c * bm, bm), :C] = x0bf_ref[pl.ds(c * bm, bm), :]

    Lb = L_ref[...].astype(jnp.bfloat16)
    rhs = xcat[pl.ds(c * bm, bm), :]
    prod = jnp.dot(Lb, rhs, preferred_element_type=jnp.float32)

    @pl.when(jb == 0)
    def _init():
        xacc[...] = prod[:, :C]
        px2_ref[...] = jnp.zeros_like(px2_ref)

    @pl.when(jb > 0)
    def _x1_accum():
        xacc[...] += prod[:, :C]

    @pl.when(c < i)
    def _lower():
        px2_ref[...] += prod[:, C:]

    @pl.when(jb == nb - 1)
    def _finish_strip():
        x1b16 = xacc[...].astype(jnp.bfloat16)
        x1f_ref[...] = xacc[...]
        xcat[pl.ds(i * bm, bm), C:] = x1b16
        px2_ref[...] = 2.0 * (
            px2_ref[...]
            + jnp.dot(Lb, x1b16, preferred_element_type=jnp.float32)
        )


def _pass2_kernel(L_ref, x1f_ref, px2_ref, x0b_ref, WA_ref, WB_ref, WC_ref,
                  bias_ref, o_ref, acc):
    nb = pl.num_programs(1)
    i = pl.program_id(0)
    jb = pl.program_id(1)
    bm = L_ref.shape[0]

    @pl.when(jb == 0)
    def _init():
        acc[...] = px2_ref[...]

    @pl.when(jb > i)
    def _accum():
        xb = x1f_ref[pl.ds(jb * bm, bm), :]
        acc[...] += 2.0 * jnp.dot(L_ref[...], xb,
                                  preferred_element_type=jnp.float32)

    @pl.when(jb == nb - 1)
    def _finish():
        x1b = x1f_ref[pl.ds(i * bm, bm), :]
        o_ref[...] = (
            jnp.dot(x0b_ref[...], WA_ref[...],
                    preferred_element_type=jnp.float32)
            + jnp.dot(x1b, WB_ref[...], preferred_element_type=jnp.float32)
            + jnp.dot(acc[...], WC_ref[...],
                      preferred_element_type=jnp.float32)
            + bias_ref[...]
        )


def kernel(x, L, weight, bias):
    N, M, Fin = x.shape
    Fout = weight.shape[1]
    x0 = jnp.transpose(x, (1, 0, 2)).reshape(M, N * Fin)
    x0bf = x0.astype(jnp.bfloat16)

    W = weight.reshape(Fin, 3, Fout)
    eyeN = jnp.eye(N, dtype=weight.dtype)
    W0e = jnp.kron(eyeN, W[:, 0, :])
    W1e = jnp.kron(eyeN, W[:, 1, :])
    W2e = jnp.kron(eyeN, W[:, 2, :])
    WA = W0e - W2e
    WB = W1e
    WC = W2e
    bias_row = jnp.tile(bias, N).reshape(1, N * Fout)

    bm = 2048
    nb = M // bm
    C = N * Fin
    Co = N * Fout

    x1f, px2 = pl.pallas_call(
        _pass1_kernel,
        grid=(nb, nb),
        in_specs=[
            pl.BlockSpec((bm, bm), lambda i, jb: (i, _p1_col(i, jb, nb))),
            pl.BlockSpec((M, C), lambda i, jb: (0, 0)),
        ],
        out_specs=[
            pl.BlockSpec((bm, C), lambda i, jb: (i, 0)),
            pl.BlockSpec((bm, C), lambda i, jb: (i, 0)),
        ],
        out_shape=[
            jax.ShapeDtypeStruct((M, C), jnp.float32),
            jax.ShapeDtypeStruct((M, C), jnp.float32),
        ],
        scratch_shapes=[
            pltpu.VMEM((M, 2 * C), jnp.bfloat16),
            pltpu.VMEM((bm, C), jnp.float32),
        ],
    )(L, x0bf)

    def _L_index(i, jb):
        row = jnp.minimum(i, nb - 2)
        col = jnp.where(i == nb - 1, nb - 1,
                        jnp.minimum(jnp.maximum(jb, i + 1), nb - 1))
        return (row, col)

    out_flat = pl.pallas_call(
        _pass2_kernel,
        grid=(nb, nb),
        in_specs=[
            pl.BlockSpec((bm, bm), _L_index),
            pl.BlockSpec((M, C), lambda i, jb: (0, 0)),
            pl.BlockSpec((bm, C), lambda i, jb: (i, 0)),
            pl.BlockSpec((bm, C), lambda i, jb: (i, 0)),
            pl.BlockSpec((C, Co), lambda i, jb: (0, 0)),
            pl.BlockSpec((C, Co), lambda i, jb: (0, 0)),
            pl.BlockSpec((C, Co), lambda i, jb: (0, 0)),
            pl.BlockSpec((1, Co), lambda i, jb: (0, 0)),
        ],
        out_specs=pl.BlockSpec((bm, Co), lambda i, jb: (i, 0)),
        out_shape=jax.ShapeDtypeStruct((M, Co), jnp.float32),
        scratch_shapes=[pltpu.VMEM((bm, C), jnp.float32)],
    )(L, x1f, px2, x0, WA, WB, WC, bias_row)

    return out_flat.reshape(M, N, Fout).transpose(1, 0, 2)


# speedup vs baseline: 3.6020x; 1.0190x over previous
import jax
import jax.numpy as jnp
from jax.experimental import pallas as pl
from jax.experimental.pallas import tpu as pltpu


def _p1_col(j, s, nb):
    shifted = s + (s >= j).astype(s.dtype)
    return jnp.where(s == nb - 1, j, shifted)


def _merged_kernel(L_ref, x0bf_ref, WAb_ref, WBb_ref, WC_ref, bias_ref,
                   o_ref, xcat, xacc, pacc):
    nb = pl.num_programs(0)
    j = pl.program_id(0)
    s = pl.program_id(1)
    bm = L_ref.shape[0]
    C = x0bf_ref.shape[1]

    def _combine(r):
        x0b = xcat[pl.ds(r * bm, bm), :C]
        x1b = xcat[pl.ds(r * bm, bm), C:]
        o_ref[...] = (
            jnp.dot(x0b, WAb_ref[...], preferred_element_type=jnp.float32)
            + jnp.dot(x1b, WBb_ref[...], preferred_element_type=jnp.float32)
            + jnp.dot(pacc[pl.ds(r * bm, bm), :], WC_ref[...],
                      preferred_element_type=jnp.float32)
            + bias_ref[...]
        )

    @pl.when(jnp.logical_and(j == 0, s == 0))
    def _zero_pacc():
        pacc[...] = jnp.zeros_like(pacc)

    @pl.when(s < nb)
    def _pass1_step():
        c = _p1_col(j, s, nb)

        @pl.when(j == 0)
        def _seed_x0():
            xcat[pl.ds(c * bm, bm), :C] = x0bf_ref[pl.ds(c * bm, bm), :]

        Lb = L_ref[...].astype(jnp.bfloat16)
        rhs = xcat[pl.ds(c * bm, bm), :]
        prod = jnp.dot(Lb, rhs, preferred_element_type=jnp.float32)

        @pl.when(s == 0)
        def _x1_init():
            xacc[...] = prod[:, :C]

        @pl.when(s > 0)
        def _x1_accum():
            xacc[...] += prod[:, :C]

        @pl.when(c < j)
        def _lower():
            pacc[pl.ds(j * bm, bm), :] += prod[:, C:]

        @pl.when(s == nb - 1)
        def _diag_finish():
            x1b16 = xacc[...].astype(jnp.bfloat16)
            xcat[pl.ds(j * bm, bm), C:] = x1b16
            pacc[pl.ds(j * bm, bm), :] += jnp.dot(
                Lb, x1b16, preferred_element_type=jnp.float32)

            @pl.when(j == nb - 1)
            def _finalize_last_strip():
                _combine(nb - 1)

    @pl.when(s >= nb)
    def _upper_step():
        i = s - nb

        @pl.when(i < j)
        def _accum_upper():
            x1j = xcat[pl.ds(j * bm, bm), C:]
            pacc[pl.ds(i * bm, bm), :] += jnp.dot(
                L_ref[...].astype(jnp.bfloat16), x1j,
                preferred_element_type=jnp.float32)

            @pl.when(j == nb - 1)
            def _finalize_strip_i():
                _combine(i)


def kernel(x, L, weight, bias):
    N, M, Fin = x.shape
    Fout = weight.shape[1]
    x0 = jnp.transpose(x, (1, 0, 2)).reshape(M, N * Fin)
    x0bf = x0.astype(jnp.bfloat16)

    W = weight.reshape(Fin, 3, Fout)
    eyeN = jnp.eye(N, dtype=weight.dtype)
    W0e = jnp.kron(eyeN, W[:, 0, :])
    W1e = jnp.kron(eyeN, W[:, 1, :])
    W2e = jnp.kron(eyeN, W[:, 2, :])
    WAb = (W0e - W2e).astype(jnp.bfloat16)
    WBb = W1e.astype(jnp.bfloat16)
    WC = 2.0 * W2e
    bias_row = jnp.tile(bias, N).reshape(1, N * Fout)

    bm = 2048
    nb = M // bm
    C = N * Fin
    Co = N * Fout

    def _L_index(j, s):
        i = s - nb
        p1 = (j, _p1_col(j, s, nb))
        up_row = jnp.where(i < j, i, jnp.maximum(j - 1, 0))
        up_col = jnp.where(j > 0, j, 0)
        return (jnp.where(s < nb, p1[0], up_row),
                jnp.where(s < nb, p1[1], up_col))

    def _o_index(j, s):
        return (jnp.where(jnp.logical_or(j < nb - 1, s < nb),
                          nb - 1, s - nb), 0)

    out_flat = pl.pallas_call(
        _merged_kernel,
        grid=(nb, 2 * nb - 1),
        in_specs=[
            pl.BlockSpec((bm, bm), _L_index),
            pl.BlockSpec((M, C), lambda j, s: (0, 0)),
            pl.BlockSpec((C, Co), lambda j, s: (0, 0)),
            pl.BlockSpec((C, Co), lambda j, s: (0, 0)),
            pl.BlockSpec((C, Co), lambda j, s: (0, 0)),
            pl.BlockSpec((1, Co), lambda j, s: (0, 0)),
        ],
        out_specs=pl.BlockSpec((bm, Co), _o_index),
        out_shape=jax.ShapeDtypeStruct((M, Co), jnp.float32),
        scratch_shapes=[
            pltpu.VMEM((M, 2 * C), jnp.bfloat16),
            pltpu.VMEM((bm, C), jnp.float32),
            pltpu.VMEM((M, C), jnp.float32),
        ],
    )(L, x0bf, WAb, WBb, WC, bias_row)

    return out_flat.reshape(M, N, Fout).transpose(1, 0, 2)
